# initial kernel scaffold (unmeasured)
import jax
import jax.numpy as jnp
from jax import lax
from jax.experimental import pallas as pl
from jax.experimental.pallas import tpu as pltpu

N_RING = 8
BLK = 1024
M_HALF = 1024


def _ring_pos(y, z):
    return jnp.where(y == 0, z, 2 * N_RING // 2 - 1 - z)


def _ring_to_yz(r):
    y = jnp.where(r >= N_RING // 2, 1, 0)
    z = jnp.where(r < N_RING // 2, r, N_RING - 1 - r)
    return y, z


def kernel(x, dy):
    k, m = x.shape
    _, f = dy.shape

    my_y = lax.axis_index("y")
    my_z = lax.axis_index("z")
    r = _ring_pos(my_y, my_z)
    dy_blk = lax.dynamic_slice(dy, (0, r * BLK), (k, BLK))

    def body(x_ref, dy_ref, out_ref, bsend, brecv, comm,
             xsend_sem, xrecv_sem, send_sems, recv_sems):
        my_x = lax.axis_index("x")
        my_y = lax.axis_index("y")
        my_z = lax.axis_index("z")
        r = _ring_pos(my_y, my_z)
        ry, rz = _ring_to_yz((r + 1) % N_RING)
        ly, lz = _ring_to_yz((r - 1) % N_RING)

        barrier = pltpu.get_barrier_semaphore()
        for tgt in [(1 - my_x, my_y, my_z), (my_x, ry, rz), (my_x, ly, lz)]:
            pl.semaphore_signal(barrier, inc=1, device_id=tgt,
                                device_id_type=pl.DeviceIdType.MESH)
        pl.semaphore_wait(barrier, 3)

        dyb = dy_ref[:, :].astype(jnp.bfloat16)

        other0 = (1 - my_x) * M_HALF
        xb_other = x_ref[:, pl.ds(other0, M_HALF)].astype(jnp.bfloat16)
        b = lax.dot_general(xb_other, dyb, (((0,), (0,)), ((), ())),
                            preferred_element_type=jnp.float32)
        bsend[:, :] = b.astype(jnp.bfloat16)

        rdma_x = pltpu.make_async_remote_copy(
            src_ref=bsend, dst_ref=brecv,
            send_sem=xsend_sem, recv_sem=xrecv_sem,
            device_id=(1 - my_x, my_y, my_z),
            device_id_type=pl.DeviceIdType.MESH,
        )
        rdma_x.start()

        mine0 = my_x * M_HALF
        xb_mine = x_ref[:, pl.ds(mine0, M_HALF)].astype(jnp.bfloat16)
        a = lax.dot_general(xb_mine, dyb, (((0,), (0,)), ((), ())),
                            preferred_element_type=jnp.float32)

        rdma_x.wait()
        own = a + brecv[:, :].astype(jnp.float32)

        out_ref[:, pl.ds(r * BLK, BLK)] = own
        comm[0, :, :] = own.astype(jnp.bfloat16)

        for h in range(N_RING - 1):
            send_slot = h % 2
            recv_slot = (h + 1) % 2
            rdma = pltpu.make_async_remote_copy(
                src_ref=comm.at[send_slot],
                dst_ref=comm.at[recv_slot],
                send_sem=send_sems.at[send_slot],
                recv_sem=recv_sems.at[recv_slot],
                device_id=(my_x, ry, rz),
                device_id_type=pl.DeviceIdType.MESH,
            )
            rdma.start()
            rdma.wait()
            origin = (r - h - 1) % N_RING
            out_ref[:, pl.ds(origin * BLK, BLK)] = (
                comm[recv_slot, :, :].astype(jnp.float32))

    return pl.pallas_call(
        body,
        out_shape=jax.ShapeDtypeStruct((M_HALF, f), jnp.float32),
        in_specs=[
            pl.BlockSpec(memory_space=pltpu.VMEM),
            pl.BlockSpec(memory_space=pltpu.VMEM),
        ],
        out_specs=pl.BlockSpec(memory_space=pltpu.VMEM),
        scratch_shapes=[
            pltpu.VMEM((M_HALF, BLK), jnp.bfloat16),
            pltpu.VMEM((M_HALF, BLK), jnp.bfloat16),
            pltpu.VMEM((2, M_HALF, BLK), jnp.bfloat16),
            pltpu.SemaphoreType.DMA,
            pltpu.SemaphoreType.DMA,
            pltpu.SemaphoreType.DMA((2,)),
            pltpu.SemaphoreType.DMA((2,)),
        ],
        compiler_params=pltpu.CompilerParams(collective_id=0),
    )(x, dy_blk)


# baseline (device time: 245663 ns/iter reference)
import jax
import jax.numpy as jnp
from jax import lax
from jax.experimental import pallas as pl
from jax.experimental.pallas import tpu as pltpu

N_RING = 8
BLK = 1024
M_HALF = 1024


def _ring_pos(y, z):
    return jnp.where(y == 0, z, 2 * N_RING // 2 - 1 - z)


def _ring_to_yz(r):
    y = jnp.where(r >= N_RING // 2, 1, 0)
    z = jnp.where(r < N_RING // 2, r, N_RING - 1 - r)
    return y, z


def kernel(x, dy):
    k, m = x.shape
    _, f = dy.shape

    my_y = lax.axis_index("y")
    my_z = lax.axis_index("z")
    r = _ring_pos(my_y, my_z)
    dy_blk = lax.dynamic_slice(dy, (0, r * BLK), (k, BLK)).astype(jnp.bfloat16)
    xb = x.astype(jnp.bfloat16)

    def body(x_ref, dy_ref, out_ref, stage, bsend, brecv, comm,
             copy_sem, xsend_sem, xrecv_sem, send_sems, recv_sems):
        my_x = lax.axis_index("x")
        my_y = lax.axis_index("y")
        my_z = lax.axis_index("z")
        r = _ring_pos(my_y, my_z)
        ry, rz = _ring_to_yz((r + 1) % N_RING)
        ly, lz = _ring_to_yz((r - 1) % N_RING)

        barrier = pltpu.get_barrier_semaphore()
        for tgt in [(1 - my_x, my_y, my_z), (my_x, ry, rz), (my_x, ly, lz)]:
            pl.semaphore_signal(barrier, inc=1, device_id=tgt,
                                device_id_type=pl.DeviceIdType.MESH)
        pl.semaphore_wait(barrier, 3)

        dyb = dy_ref[:, :]

        other0 = (1 - my_x) * M_HALF
        xb_other = x_ref[:, pl.ds(other0, M_HALF)]
        b = lax.dot_general(xb_other, dyb, (((0,), (0,)), ((), ())),
                            preferred_element_type=jnp.float32)
        bsend[:, :] = b.astype(jnp.bfloat16)

        rdma_x = pltpu.make_async_remote_copy(
            src_ref=bsend, dst_ref=brecv,
            send_sem=xsend_sem, recv_sem=xrecv_sem,
            device_id=(1 - my_x, my_y, my_z),
            device_id_type=pl.DeviceIdType.MESH,
        )
        rdma_x.start()

        mine0 = my_x * M_HALF
        xb_mine = x_ref[:, pl.ds(mine0, M_HALF)]
        a = lax.dot_general(xb_mine, dyb, (((0,), (0,)), ((), ())),
                            preferred_element_type=jnp.float32)

        rdma_x.wait()
        own = a + brecv[:, :].astype(jnp.float32)

        stage[:, :] = own
        comm[0, :, :] = own.astype(jnp.bfloat16)
        out_copy = pltpu.make_async_copy(
            stage, out_ref.at[:, pl.ds(r * BLK, BLK)], copy_sem)
        out_copy.start()

        for h in range(N_RING - 1):
            send_slot = h % 2
            recv_slot = (h + 1) % 2
            rdma = pltpu.make_async_remote_copy(
                src_ref=comm.at[send_slot],
                dst_ref=comm.at[recv_slot],
                send_sem=send_sems.at[send_slot],
                recv_sem=recv_sems.at[recv_slot],
                device_id=(my_x, ry, rz),
                device_id_type=pl.DeviceIdType.MESH,
            )
            rdma.start()
            rdma.wait()
            origin = (r - h - 1) % N_RING
            blk = comm[recv_slot, :, :].astype(jnp.float32)
            out_copy.wait()
            stage[:, :] = blk
            out_copy = pltpu.make_async_copy(
                stage, out_ref.at[:, pl.ds(origin * BLK, BLK)], copy_sem)
            out_copy.start()
        out_copy.wait()

    return pl.pallas_call(
        body,
        out_shape=jax.ShapeDtypeStruct((M_HALF, f), jnp.float32),
        in_specs=[
            pl.BlockSpec(memory_space=pltpu.VMEM),
            pl.BlockSpec(memory_space=pltpu.VMEM),
        ],
        out_specs=pl.BlockSpec(memory_space=pl.ANY),
        scratch_shapes=[
            pltpu.VMEM((M_HALF, BLK), jnp.float32),
            pltpu.VMEM((M_HALF, BLK), jnp.bfloat16),
            pltpu.VMEM((M_HALF, BLK), jnp.bfloat16),
            pltpu.VMEM((2, M_HALF, BLK), jnp.bfloat16),
            pltpu.SemaphoreType.DMA,
            pltpu.SemaphoreType.DMA,
            pltpu.SemaphoreType.DMA,
            pltpu.SemaphoreType.DMA((2,)),
            pltpu.SemaphoreType.DMA((2,)),
        ],
        compiler_params=pltpu.CompilerParams(collective_id=0),
    )(xb, dy_blk)


# device time: 165355 ns/iter; 1.4857x vs baseline; 1.4857x over previous
import jax
import jax.numpy as jnp
from jax import lax
from jax.experimental import pallas as pl
from jax.experimental.pallas import tpu as pltpu

N_RING = 8
BLK = 1024
HALF = BLK // 2
M_HALF = 1024
NSLOT = 4


def _ring_pos(y, z):
    return jnp.where(y == 0, z, N_RING - 1 - z)


def _ring_to_yz(r):
    y = jnp.where(r >= N_RING // 2, 1, 0)
    z = jnp.where(r < N_RING // 2, r, N_RING - 1 - r)
    return y, z


def kernel(x, dy):
    k, m = x.shape
    _, f = dy.shape

    my_y = lax.axis_index("y")
    my_z = lax.axis_index("z")
    r = _ring_pos(my_y, my_z)
    dy_blk = lax.dynamic_slice(dy, (0, r * BLK), (k, BLK)).astype(jnp.bfloat16)
    xb = x.astype(jnp.bfloat16)

    def body(x_ref, dy_ref, out_ref, stage_own, stage_cw, stage_ccw,
             bsend, brecv, comm_cw, comm_ccw,
             own_copy_sem, cw_copy_sem, ccw_copy_sem,
             xsend_sem, xrecv_sem,
             cw_send_sems, cw_recv_sems, ccw_send_sems, ccw_recv_sems):
        my_x = lax.axis_index("x")
        my_y = lax.axis_index("y")
        my_z = lax.axis_index("z")
        r = _ring_pos(my_y, my_z)
        ry, rz = _ring_to_yz((r + 1) % N_RING)
        ly, lz = _ring_to_yz((r - 1) % N_RING)
        right = (my_x, ry, rz)
        left = (my_x, ly, lz)

        barrier = pltpu.get_barrier_semaphore()
        for tgt in [(1 - my_x, my_y, my_z), right, left]:
            pl.semaphore_signal(barrier, inc=1, device_id=tgt,
                                device_id_type=pl.DeviceIdType.MESH)
        pl.semaphore_wait(barrier, 3)

        dyb = dy_ref[:, :]

        other0 = (1 - my_x) * M_HALF
        b = lax.dot_general(x_ref[:, pl.ds(other0, M_HALF)], dyb,
                            (((0,), (0,)), ((), ())),
                            preferred_element_type=jnp.float32)
        bsend[:, :] = b.astype(jnp.bfloat16)

        rdma_x = pltpu.make_async_remote_copy(
            src_ref=bsend, dst_ref=brecv,
            send_sem=xsend_sem, recv_sem=xrecv_sem,
            device_id=(1 - my_x, my_y, my_z),
            device_id_type=pl.DeviceIdType.MESH,
        )
        rdma_x.start()

        mine0 = my_x * M_HALF
        a = lax.dot_general(x_ref[:, pl.ds(mine0, M_HALF)], dyb,
                            (((0,), (0,)), ((), ())),
                            preferred_element_type=jnp.float32)

        rdma_x.wait()
        own = a + brecv[:, :].astype(jnp.float32)

        comm_cw[0, :, :] = own[:, :HALF].astype(jnp.bfloat16)
        comm_ccw[0, :, :] = own[:, HALF:].astype(jnp.bfloat16)
        stage_own[:, :] = own
        own_copy = pltpu.make_async_copy(
            stage_own, out_ref.at[:, pl.ds(r * BLK, BLK)], own_copy_sem)
        own_copy.start()

        def ring_desc(comm, send_sems, recv_sems, s, rv, tgt):
            return pltpu.make_async_remote_copy(
                src_ref=comm.at[s], dst_ref=comm.at[rv],
                send_sem=send_sems.at[s], recv_sem=recv_sems.at[rv],
                device_id=tgt, device_id_type=pl.DeviceIdType.MESH,
            )

        prev_cw_copy = None
        prev_ccw_copy = None
        for h in range(N_RING - 1):
            s = h % NSLOT
            rv = (h + 1) % NSLOT
            cw = ring_desc(comm_cw, cw_send_sems, cw_recv_sems, s, rv, right)
            ccw = ring_desc(comm_ccw, ccw_send_sems, ccw_recv_sems, s, rv, left)
            if h >= NSLOT:
                cw.wait_send()
                ccw.wait_send()
            cw.start()
            ccw.start()
            if h >= 1:
                o_cw = (r - h) % N_RING
                o_ccw = (r + h) % N_RING
                if prev_cw_copy is not None:
                    prev_cw_copy.wait()
                    prev_ccw_copy.wait()
                stage_cw[:, :] = comm_cw[s, :, :].astype(jnp.float32)
                stage_ccw[:, :] = comm_ccw[s, :, :].astype(jnp.float32)
                prev_cw_copy = pltpu.make_async_copy(
                    stage_cw, out_ref.at[:, pl.ds(o_cw * BLK, HALF)],
                    cw_copy_sem)
                prev_ccw_copy = pltpu.make_async_copy(
                    stage_ccw, out_ref.at[:, pl.ds(o_ccw * BLK + HALF, HALF)],
                    ccw_copy_sem)
                prev_cw_copy.start()
                prev_ccw_copy.start()
            cw.wait_recv()
            ccw.wait_recv()

        s = (N_RING - 1) % NSLOT
        o_cw = (r + 1) % N_RING
        o_ccw = (r - 1) % N_RING
        prev_cw_copy.wait()
        prev_ccw_copy.wait()
        stage_cw[:, :] = comm_cw[s, :, :].astype(jnp.float32)
        stage_ccw[:, :] = comm_ccw[s, :, :].astype(jnp.float32)
        last_cw = pltpu.make_async_copy(
            stage_cw, out_ref.at[:, pl.ds(o_cw * BLK, HALF)], cw_copy_sem)
        last_ccw = pltpu.make_async_copy(
            stage_ccw, out_ref.at[:, pl.ds(o_ccw * BLK + HALF, HALF)],
            ccw_copy_sem)
        last_cw.start()
        last_ccw.start()

        for h in range(N_RING - 1 - NSLOT, N_RING - 1):
            if h < 0:
                continue
            s = h % NSLOT
            ring_desc(comm_cw, cw_send_sems, cw_recv_sems, s, s,
                      right).wait_send()
            ring_desc(comm_ccw, ccw_send_sems, ccw_recv_sems, s, s,
                      left).wait_send()

        own_copy.wait()
        last_cw.wait()
        last_ccw.wait()

    return pl.pallas_call(
        body,
        out_shape=jax.ShapeDtypeStruct((M_HALF, f), jnp.float32),
        in_specs=[
            pl.BlockSpec(memory_space=pltpu.VMEM),
            pl.BlockSpec(memory_space=pltpu.VMEM),
        ],
        out_specs=pl.BlockSpec(memory_space=pl.ANY),
        scratch_shapes=[
            pltpu.VMEM((M_HALF, BLK), jnp.float32),
            pltpu.VMEM((M_HALF, HALF), jnp.float32),
            pltpu.VMEM((M_HALF, HALF), jnp.float32),
            pltpu.VMEM((M_HALF, BLK), jnp.bfloat16),
            pltpu.VMEM((M_HALF, BLK), jnp.bfloat16),
            pltpu.VMEM((NSLOT, M_HALF, HALF), jnp.bfloat16),
            pltpu.VMEM((NSLOT, M_HALF, HALF), jnp.bfloat16),
            pltpu.SemaphoreType.DMA,
            pltpu.SemaphoreType.DMA,
            pltpu.SemaphoreType.DMA,
            pltpu.SemaphoreType.DMA,
            pltpu.SemaphoreType.DMA,
            pltpu.SemaphoreType.DMA((NSLOT,)),
            pltpu.SemaphoreType.DMA((NSLOT,)),
            pltpu.SemaphoreType.DMA((NSLOT,)),
            pltpu.SemaphoreType.DMA((NSLOT,)),
        ],
        compiler_params=pltpu.CompilerParams(collective_id=0),
    )(xb, dy_blk)


# device time: 160276 ns/iter; 1.5327x vs baseline; 1.0317x over previous
import os

import jax
import jax.numpy as jnp
from jax import lax
from jax.experimental import pallas as pl
from jax.experimental.pallas import tpu as pltpu

_KPHASES = int(os.environ.get("KPHASES", "3"))

N_RING = 8
BLK = 1024
HALF = BLK // 2
M_HALF = 1024
NSLOT = 4


def _ring_pos(y, z):
    return jnp.where(y == 0, z, N_RING - 1 - z)


def _ring_to_yz(r):
    y = jnp.where(r >= N_RING // 2, 1, 0)
    z = jnp.where(r < N_RING // 2, r, N_RING - 1 - r)
    return y, z


def kernel(x, dy):
    k, m = x.shape
    _, f = dy.shape

    my_y = lax.axis_index("y")
    my_z = lax.axis_index("z")
    r = _ring_pos(my_y, my_z)
    dy_blk = lax.dynamic_slice(dy, (0, r * BLK), (k, BLK)).astype(jnp.bfloat16)
    xb = x.astype(jnp.bfloat16)

    def body(x_ref, dy_ref, out_ref, bsend, brecv, comm_cw, comm_ccw,
             cw_out_sem, ccw_out_sem, xsend_sem, xrecv_sem,
             cw_send_sems, cw_recv_sems, ccw_send_sems, ccw_recv_sems):
        my_x = lax.axis_index("x")
        my_y = lax.axis_index("y")
        my_z = lax.axis_index("z")
        r = _ring_pos(my_y, my_z)
        ry, rz = _ring_to_yz((r + 1) % N_RING)
        ly, lz = _ring_to_yz((r - 1) % N_RING)
        right = (my_x, ry, rz)
        left = (my_x, ly, lz)

        if _KPHASES >= 2:
            barrier = pltpu.get_barrier_semaphore()
            for tgt in [(1 - my_x, my_y, my_z), right, left]:
                pl.semaphore_signal(barrier, inc=1, device_id=tgt,
                                    device_id_type=pl.DeviceIdType.MESH)
            pl.semaphore_wait(barrier, 3)

        dyb = dy_ref[:, :]

        other0 = (1 - my_x) * M_HALF
        b = lax.dot_general(x_ref[:, pl.ds(other0, M_HALF)], dyb,
                            (((0,), (0,)), ((), ())),
                            preferred_element_type=jnp.float32)
        bsend[:, :] = b.astype(jnp.bfloat16)

        if _KPHASES >= 2:
            rdma_x = pltpu.make_async_remote_copy(
                src_ref=bsend, dst_ref=brecv,
                send_sem=xsend_sem, recv_sem=xrecv_sem,
                device_id=(1 - my_x, my_y, my_z),
                device_id_type=pl.DeviceIdType.MESH,
            )
            rdma_x.start()

        mine0 = my_x * M_HALF
        a = lax.dot_general(x_ref[:, pl.ds(mine0, M_HALF)], dyb,
                            (((0,), (0,)), ((), ())),
                            preferred_element_type=jnp.float32)

        if _KPHASES >= 2:
            rdma_x.wait()
            own = a + brecv[:, :].astype(jnp.float32)
        else:
            own = a + b

        comm_cw[0, :, :] = own[:, :HALF].astype(jnp.bfloat16)
        comm_ccw[0, :, :] = own[:, HALF:].astype(jnp.bfloat16)
        prev_cw_copy = pltpu.make_async_copy(
            comm_cw.at[0], out_ref.at[:, pl.ds(r * BLK, HALF)], cw_out_sem)
        prev_ccw_copy = pltpu.make_async_copy(
            comm_ccw.at[0], out_ref.at[:, pl.ds(r * BLK + HALF, HALF)],
            ccw_out_sem)
        prev_cw_copy.start()
        prev_ccw_copy.start()

        def ring_desc(comm, send_sems, recv_sems, s, rv, tgt):
            return pltpu.make_async_remote_copy(
                src_ref=comm.at[s], dst_ref=comm.at[rv],
                send_sem=send_sems.at[s], recv_sem=recv_sems.at[rv],
                device_id=tgt, device_id_type=pl.DeviceIdType.MESH,
            )

        for h in range(N_RING - 1 if _KPHASES >= 3 else 0):
            s = h % NSLOT
            rv = (h + 1) % NSLOT
            cw = ring_desc(comm_cw, cw_send_sems, cw_recv_sems, s, rv, right)
            ccw = ring_desc(comm_ccw, ccw_send_sems, ccw_recv_sems, s, rv, left)
            if h >= NSLOT:
                cw.wait_send()
                ccw.wait_send()
            cw.start()
            ccw.start()
            if h >= 1:
                o_cw = (r - h) % N_RING
                o_ccw = (r + h) % N_RING
                prev_cw_copy.wait()
                prev_ccw_copy.wait()
                prev_cw_copy = pltpu.make_async_copy(
                    comm_cw.at[s], out_ref.at[:, pl.ds(o_cw * BLK, HALF)],
                    cw_out_sem)
                prev_ccw_copy = pltpu.make_async_copy(
                    comm_ccw.at[s],
                    out_ref.at[:, pl.ds(o_ccw * BLK + HALF, HALF)],
                    ccw_out_sem)
                prev_cw_copy.start()
                prev_ccw_copy.start()
            cw.wait_recv()
            ccw.wait_recv()

        if _KPHASES < 3:
            prev_cw_copy.wait()
            prev_ccw_copy.wait()
            return

        s = (N_RING - 1) % NSLOT
        o_cw = (r + 1) % N_RING
        o_ccw = (r - 1) % N_RING
        prev_cw_copy.wait()
        prev_ccw_copy.wait()
        last_cw = pltpu.make_async_copy(
            comm_cw.at[s], out_ref.at[:, pl.ds(o_cw * BLK, HALF)], cw_out_sem)
        last_ccw = pltpu.make_async_copy(
            comm_ccw.at[s], out_ref.at[:, pl.ds(o_ccw * BLK + HALF, HALF)],
            ccw_out_sem)
        last_cw.start()
        last_ccw.start()

        for h in range(N_RING - 1 - NSLOT, N_RING - 1):
            if h < 0:
                continue
            s = h % NSLOT
            ring_desc(comm_cw, cw_send_sems, cw_recv_sems, s, s,
                      right).wait_send()
            ring_desc(comm_ccw, ccw_send_sems, ccw_recv_sems, s, s,
                      left).wait_send()

        last_cw.wait()
        last_ccw.wait()

    out_bf = pl.pallas_call(
        body,
        out_shape=jax.ShapeDtypeStruct((M_HALF, f), jnp.bfloat16),
        in_specs=[
            pl.BlockSpec(memory_space=pltpu.VMEM),
            pl.BlockSpec(memory_space=pltpu.VMEM),
        ],
        out_specs=pl.BlockSpec(memory_space=pl.ANY),
        scratch_shapes=[
            pltpu.VMEM((M_HALF, BLK), jnp.bfloat16),
            pltpu.VMEM((M_HALF, BLK), jnp.bfloat16),
            pltpu.VMEM((NSLOT, M_HALF, HALF), jnp.bfloat16),
            pltpu.VMEM((NSLOT, M_HALF, HALF), jnp.bfloat16),
            pltpu.SemaphoreType.DMA,
            pltpu.SemaphoreType.DMA,
            pltpu.SemaphoreType.DMA,
            pltpu.SemaphoreType.DMA,
            pltpu.SemaphoreType.DMA((NSLOT,)),
            pltpu.SemaphoreType.DMA((NSLOT,)),
            pltpu.SemaphoreType.DMA((NSLOT,)),
            pltpu.SemaphoreType.DMA((NSLOT,)),
        ],
        compiler_params=(pltpu.CompilerParams(collective_id=0)
                         if _KPHASES >= 2 else pltpu.CompilerParams()),
    )(xb, dy_blk)
    return out_bf.astype(jnp.float32)


# device time: 138247 ns/iter; 1.7770x vs baseline; 1.1593x over previous
import os

import jax
import jax.numpy as jnp
from jax import lax
from jax.experimental import pallas as pl
from jax.experimental.pallas import tpu as pltpu

_KPHASES = int(os.environ.get("KPHASES", "3"))

N_RING = 8
BLK = 1024
QRT = BLK // 4
M_HALF = 1024
NSLOT = 4


def _ring_pos(y, z):
    return jnp.where(y == 0, z, N_RING - 1 - z)


def _ring_to_yz(r):
    y = jnp.where(r >= N_RING // 2, 1, 0)
    z = jnp.where(r < N_RING // 2, r, N_RING - 1 - r)
    return y, z


def kernel(x, dy):
    k, m = x.shape
    _, f = dy.shape

    my_y = lax.axis_index("y")
    my_z = lax.axis_index("z")
    r = _ring_pos(my_y, my_z)
    dy_blk = lax.dynamic_slice(dy, (0, r * BLK), (k, BLK)).astype(jnp.bfloat16)
    xb = x.astype(jnp.bfloat16)

    def body(x_ref, dy_ref, out_ref, bsend, brecv, comm_cw, comm_ccw,
             out_sems, xsend_sems, xrecv_sems,
             cw_send_sems, cw_recv_sems, ccw_send_sems, ccw_recv_sems):
        my_x = lax.axis_index("x")
        my_y = lax.axis_index("y")
        my_z = lax.axis_index("z")
        r = _ring_pos(my_y, my_z)
        ry, rz = _ring_to_yz((r + 1) % N_RING)
        ly, lz = _ring_to_yz((r - 1) % N_RING)
        right = (my_x, ry, rz)
        left = (my_x, ly, lz)

        if _KPHASES >= 2:
            barrier = pltpu.get_barrier_semaphore()
            for tgt in [(1 - my_x, my_y, my_z), right, left]:
                pl.semaphore_signal(barrier, inc=1, device_id=tgt,
                                    device_id_type=pl.DeviceIdType.MESH)
            pl.semaphore_wait(barrier, 3)

        dyb = dy_ref[:, :]

        lanes = [
            dict(name="cw0", cw=True, c=0, col0=0),
            dict(name="ccw0", cw=False, c=0, col0=512),
            dict(name="cw1", cw=True, c=1, col0=256),
            dict(name="ccw1", cw=False, c=1, col0=768),
        ]
        for i, ln in enumerate(lanes):
            ln["xidx"] = i
            ln["comm"] = comm_cw if ln["cw"] else comm_ccw
            ln["ss"] = cw_send_sems if ln["cw"] else ccw_send_sems
            ln["rs"] = cw_recv_sems if ln["cw"] else ccw_recv_sems
            ln["tgt"] = right if ln["cw"] else left

        other0 = (1 - my_x) * M_HALF
        b = lax.dot_general(x_ref[:, pl.ds(other0, M_HALF)], dyb,
                            (((0,), (0,)), ((), ())),
                            preferred_element_type=jnp.float32)
        for ln in lanes:
            bsend[ln["xidx"], :, :] = (
                b[:, ln["col0"]:ln["col0"] + QRT].astype(jnp.bfloat16))

        xchg = []
        if _KPHASES >= 2:
            for ln in lanes:
                i = ln["xidx"]
                d = pltpu.make_async_remote_copy(
                    src_ref=bsend.at[i], dst_ref=brecv.at[i],
                    send_sem=xsend_sems.at[i], recv_sem=xrecv_sems.at[i],
                    device_id=(1 - my_x, my_y, my_z),
                    device_id_type=pl.DeviceIdType.MESH,
                )
                d.start()
                xchg.append(d)

        mine0 = my_x * M_HALF
        a = lax.dot_general(x_ref[:, pl.ds(mine0, M_HALF)], dyb,
                            (((0,), (0,)), ((), ())),
                            preferred_element_type=jnp.float32)

        def ring_desc(ln, h):
            s = h % NSLOT
            rv = (h + 1) % NSLOT
            return pltpu.make_async_remote_copy(
                src_ref=ln["comm"].at[ln["c"], s],
                dst_ref=ln["comm"].at[ln["c"], rv],
                send_sem=ln["ss"].at[ln["c"], s],
                recv_sem=ln["rs"].at[ln["c"], rv],
                device_id=ln["tgt"], device_id_type=pl.DeviceIdType.MESH,
            )

        def out_copy(ln, slot, origin):
            return pltpu.make_async_copy(
                ln["comm"].at[ln["c"], slot],
                out_ref.at[:, pl.ds(origin * BLK + ln["col0"], QRT)],
                out_sems.at[ln["xidx"]])

        for ln in lanes:
            i = ln["xidx"]
            if _KPHASES >= 2:
                xchg[i].wait()
                own_c = (a[:, ln["col0"]:ln["col0"] + QRT]
                         + brecv[i, :, :].astype(jnp.float32))
            else:
                own_c = (a[:, ln["col0"]:ln["col0"] + QRT]
                         + b[:, ln["col0"]:ln["col0"] + QRT])
            ln["comm"][ln["c"], 0, :, :] = own_c.astype(jnp.bfloat16)
            if _KPHASES >= 3:
                d = ring_desc(ln, 0)
                d.start()
                ln["prev"] = d
            ln["prev_out"] = out_copy(ln, 0, r)
            ln["prev_out"].start()

        if _KPHASES < 3:
            for ln in lanes:
                ln["prev_out"].wait()
            return

        for h in range(1, N_RING - 1):
            s = h % NSLOT
            for ln in lanes:
                ln["prev"].wait_recv()
                d = ring_desc(ln, h)
                if h >= NSLOT:
                    d.wait_send()
                d.start()
                ln["prev"] = d
                o = (r - h) % N_RING if ln["cw"] else (r + h) % N_RING
                ln["prev_out"].wait()
                ln["prev_out"] = out_copy(ln, s, o)
                ln["prev_out"].start()

        s = (N_RING - 1) % NSLOT
        for ln in lanes:
            ln["prev"].wait_recv()
            o = (r + 1) % N_RING if ln["cw"] else (r - 1) % N_RING
            ln["prev_out"].wait()
            ln["prev_out"] = out_copy(ln, s, o)
            ln["prev_out"].start()

        for h in range(max(0, N_RING - 1 - NSLOT), N_RING - 1):
            for ln in lanes:
                ring_desc(ln, h).wait_send()

        for ln in lanes:
            ln["prev_out"].wait()

    out_bf = pl.pallas_call(
        body,
        out_shape=jax.ShapeDtypeStruct((M_HALF, f), jnp.bfloat16),
        in_specs=[
            pl.BlockSpec(memory_space=pltpu.VMEM),
            pl.BlockSpec(memory_space=pltpu.VMEM),
        ],
        out_specs=pl.BlockSpec(memory_space=pl.ANY),
        scratch_shapes=[
            pltpu.VMEM((4, M_HALF, QRT), jnp.bfloat16),
            pltpu.VMEM((4, M_HALF, QRT), jnp.bfloat16),
            pltpu.VMEM((2, NSLOT, M_HALF, QRT), jnp.bfloat16),
            pltpu.VMEM((2, NSLOT, M_HALF, QRT), jnp.bfloat16),
            pltpu.SemaphoreType.DMA((4,)),
            pltpu.SemaphoreType.DMA((4,)),
            pltpu.SemaphoreType.DMA((4,)),
            pltpu.SemaphoreType.DMA((2, NSLOT)),
            pltpu.SemaphoreType.DMA((2, NSLOT)),
            pltpu.SemaphoreType.DMA((2, NSLOT)),
            pltpu.SemaphoreType.DMA((2, NSLOT)),
        ],
        compiler_params=(pltpu.CompilerParams(collective_id=0)
                         if _KPHASES >= 2 else pltpu.CompilerParams()),
    )(xb, dy_blk)
    return out_bf.astype(jnp.float32)


# device time: 135646 ns/iter; 1.8111x vs baseline; 1.0192x over previous
import os

import jax
import jax.numpy as jnp
from jax import lax
from jax.experimental import pallas as pl
from jax.experimental.pallas import tpu as pltpu

_KPHASES = int(os.environ.get("KPHASES", "3"))

N_RING = 8
BLK = 1024
HALF = BLK // 2
QRT = 128
LPD = HALF // QRT
M_HALF = 1024
NSLOT = 4


def _ring_pos(y, z):
    return jnp.where(y == 0, z, N_RING - 1 - z)


def _ring_to_yz(r):
    y = jnp.where(r >= N_RING // 2, 1, 0)
    z = jnp.where(r < N_RING // 2, r, N_RING - 1 - r)
    return y, z


def kernel(x, dy):
    k, m = x.shape
    _, f = dy.shape

    my_y = lax.axis_index("y")
    my_z = lax.axis_index("z")
    r = _ring_pos(my_y, my_z)
    dy_blk = lax.dynamic_slice(dy, (0, r * BLK), (k, BLK)).astype(jnp.bfloat16)
    xb = x.astype(jnp.bfloat16)

    def body(x_ref, dy_ref, out_ref, bsend, brecv, comm_cw, comm_ccw,
             out_sems, xsend_sems, xrecv_sems,
             cw_send_sems, cw_recv_sems, ccw_send_sems, ccw_recv_sems):
        my_x = lax.axis_index("x")
        my_y = lax.axis_index("y")
        my_z = lax.axis_index("z")
        r = _ring_pos(my_y, my_z)
        ry, rz = _ring_to_yz((r + 1) % N_RING)
        ly, lz = _ring_to_yz((r - 1) % N_RING)
        right = (my_x, ry, rz)
        left = (my_x, ly, lz)

        if _KPHASES >= 2:
            barrier = pltpu.get_barrier_semaphore()
            for tgt in [(1 - my_x, my_y, my_z), right, left]:
                pl.semaphore_signal(barrier, inc=1, device_id=tgt,
                                    device_id_type=pl.DeviceIdType.MESH)
            pl.semaphore_wait(barrier, 3)

        dyb = dy_ref[:, :]

        lanes = []
        for c in range(LPD):
            lanes.append(dict(cw=True, c=c, col0=c * QRT))
            lanes.append(dict(cw=False, c=c, col0=HALF + c * QRT))
        for i, ln in enumerate(lanes):
            ln["xidx"] = i
            ln["comm"] = comm_cw if ln["cw"] else comm_ccw
            ln["ss"] = cw_send_sems if ln["cw"] else ccw_send_sems
            ln["rs"] = cw_recv_sems if ln["cw"] else ccw_recv_sems
            ln["tgt"] = right if ln["cw"] else left

        other0 = (1 - my_x) * M_HALF
        b = lax.dot_general(x_ref[:, pl.ds(other0, M_HALF)], dyb,
                            (((0,), (0,)), ((), ())),
                            preferred_element_type=jnp.float32)
        for ln in lanes:
            bsend[ln["xidx"], :, :] = (
                b[:, ln["col0"]:ln["col0"] + QRT].astype(jnp.bfloat16))

        xchg = []
        if _KPHASES >= 2:
            for ln in lanes:
                i = ln["xidx"]
                d = pltpu.make_async_remote_copy(
                    src_ref=bsend.at[i], dst_ref=brecv.at[i],
                    send_sem=xsend_sems.at[i], recv_sem=xrecv_sems.at[i],
                    device_id=(1 - my_x, my_y, my_z),
                    device_id_type=pl.DeviceIdType.MESH,
                )
                d.start()
                xchg.append(d)

        mine0 = my_x * M_HALF
        a = lax.dot_general(x_ref[:, pl.ds(mine0, M_HALF)], dyb,
                            (((0,), (0,)), ((), ())),
                            preferred_element_type=jnp.float32)

        def ring_desc(ln, h):
            s = h % NSLOT
            rv = (h + 1) % NSLOT
            return pltpu.make_async_remote_copy(
                src_ref=ln["comm"].at[ln["c"], s],
                dst_ref=ln["comm"].at[ln["c"], rv],
                send_sem=ln["ss"].at[ln["c"], s],
                recv_sem=ln["rs"].at[ln["c"], rv],
                device_id=ln["tgt"], device_id_type=pl.DeviceIdType.MESH,
            )

        def out_copy(ln, slot, origin):
            return pltpu.make_async_copy(
                ln["comm"].at[ln["c"], slot],
                out_ref.at[:, pl.ds(origin * BLK + ln["col0"], QRT)],
                out_sems.at[ln["xidx"]])

        for ln in lanes:
            i = ln["xidx"]
            if _KPHASES >= 2:
                xchg[i].wait()
                own_c = (a[:, ln["col0"]:ln["col0"] + QRT]
                         + brecv[i, :, :].astype(jnp.float32))
            else:
                own_c = (a[:, ln["col0"]:ln["col0"] + QRT]
                         + b[:, ln["col0"]:ln["col0"] + QRT])
            ln["comm"][ln["c"], 0, :, :] = own_c.astype(jnp.bfloat16)
            if _KPHASES >= 3:
                d = ring_desc(ln, 0)
                d.start()
                ln["prev"] = d
            ln["prev_out"] = out_copy(ln, 0, r)
            ln["prev_out"].start()

        if _KPHASES < 3:
            for ln in lanes:
                ln["prev_out"].wait()
            return

        for h in range(1, N_RING - 1):
            s = h % NSLOT
            for ln in lanes:
                ln["prev"].wait_recv()
                d = ring_desc(ln, h)
                if h >= NSLOT:
                    d.wait_send()
                d.start()
                ln["prev"] = d
                o = (r - h) % N_RING if ln["cw"] else (r + h) % N_RING
                ln["prev_out"].wait()
                ln["prev_out"] = out_copy(ln, s, o)
                ln["prev_out"].start()

        s = (N_RING - 1) % NSLOT
        for ln in lanes:
            ln["prev"].wait_recv()
            o = (r + 1) % N_RING if ln["cw"] else (r - 1) % N_RING
            ln["prev_out"].wait()
            ln["prev_out"] = out_copy(ln, s, o)
            ln["prev_out"].start()

        for h in range(max(0, N_RING - 1 - NSLOT), N_RING - 1):
            for ln in lanes:
                ring_desc(ln, h).wait_send()

        for ln in lanes:
            ln["prev_out"].wait()

    out_bf = pl.pallas_call(
        body,
        out_shape=jax.ShapeDtypeStruct((M_HALF, f), jnp.bfloat16),
        in_specs=[
            pl.BlockSpec(memory_space=pltpu.VMEM),
            pl.BlockSpec(memory_space=pltpu.VMEM),
        ],
        out_specs=pl.BlockSpec(memory_space=pl.ANY),
        scratch_shapes=[
            pltpu.VMEM((2 * LPD, M_HALF, QRT), jnp.bfloat16),
            pltpu.VMEM((2 * LPD, M_HALF, QRT), jnp.bfloat16),
            pltpu.VMEM((LPD, NSLOT, M_HALF, QRT), jnp.bfloat16),
            pltpu.VMEM((LPD, NSLOT, M_HALF, QRT), jnp.bfloat16),
            pltpu.SemaphoreType.DMA((2 * LPD,)),
            pltpu.SemaphoreType.DMA((2 * LPD,)),
            pltpu.SemaphoreType.DMA((2 * LPD,)),
            pltpu.SemaphoreType.DMA((LPD, NSLOT)),
            pltpu.SemaphoreType.DMA((LPD, NSLOT)),
            pltpu.SemaphoreType.DMA((LPD, NSLOT)),
            pltpu.SemaphoreType.DMA((LPD, NSLOT)),
        ],
        compiler_params=(pltpu.CompilerParams(collective_id=0)
                         if _KPHASES >= 2 else pltpu.CompilerParams()),
    )(xb, dy_blk)
    return out_bf.astype(jnp.float32)
